# Initial kernel scaffold; baseline (speedup 1.0000x reference)
import jax
import jax.numpy as jnp
from jax.experimental import pallas as pl


def kernel(x, edge_index, params):
    raise NotImplementedError("write your pallas kernel here")


# speedup vs baseline: 1.1698x; 1.1698x over previous
import functools

import jax
import jax.numpy as jnp
from jax import lax
from jax.experimental import pallas as pl
from jax.experimental.pallas import tpu as pltpu

N = 10000
NPAD = 10240
E = 160000
ROW_BLK = 512


def _pre_body(h_ref, wt_ref, wb_ref, b_ref, a_ref, bo_ref):
    hb = h_ref[...]
    a_ref[...] = (jnp.dot(hb, wt_ref[...], preferred_element_type=jnp.float32)
                  + b_ref[...])
    bo_ref[...] = jnp.dot(hb, wb_ref[...], preferred_element_type=jnp.float32)


def _pre_tc(h, wt, wb, b):
    f_in = h.shape[1]
    f = wt.shape[1]
    grid = NPAD // ROW_BLK
    return pl.pallas_call(
        _pre_body,
        grid=(grid,),
        in_specs=[
            pl.BlockSpec((ROW_BLK, f_in), lambda i: (i, 0)),
            pl.BlockSpec((f_in, f), lambda i: (0, 0)),
            pl.BlockSpec((f_in, f), lambda i: (0, 0)),
            pl.BlockSpec((1, f), lambda i: (0, 0)),
        ],
        out_specs=[
            pl.BlockSpec((ROW_BLK, f), lambda i: (i, 0)),
            pl.BlockSpec((ROW_BLK, f), lambda i: (i, 0)),
        ],
        out_shape=[
            jax.ShapeDtypeStruct((NPAD, f), jnp.float32),
            jax.ShapeDtypeStruct((NPAD, f), jnp.float32),
        ],
    )(h, wt, wb, b.reshape(1, f))


def _scales_body(deg_ref, sc1_ref, sc2_ref):
    deg = deg_ref[...]
    nid = lax.broadcasted_iota(jnp.int32, deg.shape, 0) * 128 + \
        lax.broadcasted_iota(jnp.int32, deg.shape, 1)
    valid = nid < N
    avg_log = jnp.sum(jnp.where(valid, jnp.log(deg + 1.0), 0.0)) / N
    scale = jnp.log(jnp.maximum(deg, 1.0) + 1.0)
    sc1_ref[...] = scale / avg_log
    sc2_ref[...] = avg_log / scale


def _scales_tc(deg):
    deg2 = deg.reshape(NPAD // 128, 128)
    sc1, sc2 = pl.pallas_call(
        _scales_body,
        out_shape=[jax.ShapeDtypeStruct(deg2.shape, jnp.float32)] * 2,
    )(deg2)
    return sc1.reshape(NPAD, 1), sc2.reshape(NPAD, 1)


def _post_body(nf_out, final, h_ref, a_ref, s1_ref, s2_ref, mn_ref, mx_ref,
               deg_ref, sc1_ref, sc2_ref, pw_ref, pb_ref, lw_ref, lb_ref,
               g_ref, be_ref, *rest):
    if final:
        cw_ref, cb_ref, res_ref = rest[0], rest[1], None
        pna_ref, bn_ref, logit_ref = rest[2], rest[3], rest[4]
    else:
        res_ref = rest[0] if len(rest) == 2 else None
        out_ref = rest[-1]
    deg = deg_ref[...]
    degc = jnp.maximum(deg, 1.0)
    a = a_ref[...]
    s1 = s1_ref[...]
    s2 = s2_ref[...]
    s = deg * a + s1
    q = deg * a * a + 2.0 * a * s1 + s2
    mean = s / degc
    var = jnp.maximum(q / degc - mean * mean, 0.0)
    std = jnp.sqrt(var + 1e-5)
    has = deg > 0.0
    mn = jnp.where(has, a + mn_ref[...], 0.0)
    mx = jnp.where(has, a + mx_ref[...], 0.0)
    aggr = jnp.concatenate([mean, mn, mx, std], axis=-1)
    cat = jnp.concatenate(
        [h_ref[...], aggr, aggr * sc1_ref[...], aggr * sc2_ref[...]], axis=-1)
    o = jnp.dot(cat, pw_ref[...], preferred_element_type=jnp.float32) + pb_ref[...]
    o = jnp.dot(o, lw_ref[...], preferred_element_type=jnp.float32) + lb_ref[...]
    mu = jnp.mean(o, axis=-1, keepdims=True)
    xc = o - mu
    v = jnp.mean(xc * xc, axis=-1, keepdims=True)
    ln = xc / jnp.sqrt(v + 1e-5) * g_ref[...] + be_ref[...]
    elu = jnp.where(ln > 0.0, ln, jnp.exp(jnp.minimum(ln, 0.0)) - 1.0)
    if final:
        pna_ref[...] = o
        bn_ref[...] = ln
        logit_ref[...] = (jnp.dot(elu, cw_ref[...],
                                  preferred_element_type=jnp.float32)
                          + cb_ref[...])
    else:
        if res_ref is not None:
            out_ref[...] = elu + res_ref[...]
        else:
            out_ref[...] = elu


def _post_tc(h, a, s1, s2, mn, mx, deg_col, sc1, sc2, conv_p, ln_p,
             residual=None, classifier=None):
    f = a.shape[1]
    f_out = conv_p['post']['W'].shape[1]
    final = classifier is not None
    grid = NPAD // ROW_BLK
    row = lambda i: (i, 0)
    fixed = lambda i: (0, 0)
    in_specs = [
        pl.BlockSpec((ROW_BLK, f), row),
        pl.BlockSpec((ROW_BLK, f), row),
        pl.BlockSpec((ROW_BLK, f), row),
        pl.BlockSpec((ROW_BLK, f), row),
        pl.BlockSpec((ROW_BLK, f), row),
        pl.BlockSpec((ROW_BLK, f), row),
        pl.BlockSpec((ROW_BLK, 1), row),
        pl.BlockSpec((ROW_BLK, 1), row),
        pl.BlockSpec((ROW_BLK, 1), row),
        pl.BlockSpec((13 * f, f_out), fixed),
        pl.BlockSpec((1, f_out), fixed),
        pl.BlockSpec((f_out, f_out), fixed),
        pl.BlockSpec((1, f_out), fixed),
        pl.BlockSpec((1, f_out), fixed),
        pl.BlockSpec((1, f_out), fixed),
    ]
    args = [h, a, s1, s2, mn, mx, deg_col, sc1, sc2,
            conv_p['post']['W'], conv_p['post']['b'].reshape(1, f_out),
            conv_p['lin']['W'], conv_p['lin']['b'].reshape(1, f_out),
            ln_p['gamma'].reshape(1, f_out), ln_p['beta'].reshape(1, f_out)]
    if final:
        ncls = classifier['W'].shape[1]
        in_specs += [pl.BlockSpec((f_out, ncls), fixed),
                     pl.BlockSpec((1, ncls), fixed)]
        args += [classifier['W'], classifier['b'].reshape(1, ncls)]
        out_specs = [pl.BlockSpec((ROW_BLK, f_out), row),
                     pl.BlockSpec((ROW_BLK, f_out), row),
                     pl.BlockSpec((ROW_BLK, ncls), row)]
        out_shape = [jax.ShapeDtypeStruct((NPAD, f_out), jnp.float32),
                     jax.ShapeDtypeStruct((NPAD, f_out), jnp.float32),
                     jax.ShapeDtypeStruct((NPAD, ncls), jnp.float32)]
    else:
        if residual is not None:
            in_specs.append(pl.BlockSpec((ROW_BLK, f_out), row))
            args.append(residual)
        out_specs = pl.BlockSpec((ROW_BLK, f_out), row)
        out_shape = jax.ShapeDtypeStruct((NPAD, f_out), jnp.float32)
    body = functools.partial(_post_body, f_out, final)
    return pl.pallas_call(
        body, grid=(grid,), in_specs=in_specs, out_specs=out_specs,
        out_shape=out_shape,
    )(*args)


def _segment_stats(b_mat, src, dst):
    bs = b_mat[src]
    s1 = jax.ops.segment_sum(bs, dst, num_segments=NPAD)
    s2 = jax.ops.segment_sum(bs * bs, dst, num_segments=NPAD)
    mn = jax.ops.segment_min(bs, dst, num_segments=NPAD)
    mn = jnp.where(jnp.isfinite(mn), mn, 0.0)
    mx = jax.ops.segment_max(bs, dst, num_segments=NPAD)
    mx = jnp.where(jnp.isfinite(mx), mx, 0.0)
    return s1, s2, mn, mx


def kernel(x, edge_index, params):
    src, dst = edge_index[0], edge_index[1]
    xp = jnp.pad(x, ((0, NPAD - N), (0, 0)))
    deg = jax.ops.segment_sum(jnp.ones((E,), jnp.float32), dst,
                              num_segments=NPAD)
    sc1, sc2 = _scales_tc(deg)
    deg_col = deg.reshape(NPAD, 1)

    def layer(h, cp, lnp, residual=None, classifier=None):
        f_in = h.shape[1]
        a, b_mat = _pre_tc(h, cp['pre']['W'][:f_in], cp['pre']['W'][f_in:],
                           cp['pre']['b'])
        s1, s2, mn, mx = _segment_stats(b_mat, src, dst)
        return _post_tc(h, a, s1, s2, mn, mx, deg_col, sc1, sc2, cp, lnp,
                        residual=residual, classifier=classifier)

    p = params
    h1 = layer(xp, p['conv1'], p['bn1'])
    h2 = layer(h1, p['conv2'], p['bn2'])
    h4_in = layer(h2, p['conv3'], p['bn3'], residual=h1)
    out_pna, out_bn, logits = layer(h4_in, p['conv4'], p['bn4'],
                                    classifier=p['classifier'])
    return (logits[:N], out_pna[:N], out_bn[:N])
